# baseline (speedup 1.0000x reference)
import numpy as np
import jax
import jax.numpy as jnp
from jax.experimental import pallas as pl
from jax.experimental.pallas import tpu as pltpu


def _dct_mat(N):
    n = np.arange(N, dtype=np.float64)
    k = np.arange(N, dtype=np.float64)[:, None]
    M = np.cos(np.pi * (n + 0.5) * k / N)
    scale = np.where(k == 0, np.sqrt(1.0 / N), np.sqrt(2.0 / N))
    return M * scale


_A_NP = (_dct_mat(32).T @ _dct_mat(64)[:32, :]).astype(np.float32)


def _pool_body(a_ref, x_ref, o_ref):
    A = a_ref[...]
    X = x_ref[0]
    t = jnp.dot(A, X, preferred_element_type=jnp.float32).reshape(32, 64, 64)
    o = jnp.stack([jnp.dot(A, t[k], preferred_element_type=jnp.float32)
                   for k in range(32)])
    o_ref[0] = o[:, :, :32]


def kernel(x):
    B, C, D, H, W = x.shape
    BC = B * C
    xr = x.reshape(BC, D, H * W)
    A = jnp.asarray(_A_NP)

    out = pl.pallas_call(
        _pool_body,
        grid=(BC,),
        in_specs=[
            pl.BlockSpec((32, 64), lambda i: (0, 0)),
            pl.BlockSpec((1, 64, 4096), lambda i: (i, 0, 0)),
        ],
        out_specs=pl.BlockSpec((1, 32, 32, 32), lambda i: (i, 0, 0, 0)),
        out_shape=jax.ShapeDtypeStruct((BC, 32, 32, 32), jnp.float32),
        compiler_params=pltpu.CompilerParams(
            dimension_semantics=("parallel",),
        ),
    )(A, xr)
    return out.reshape(B, C, 32, 32, 32)


# speedup vs baseline: 1.9367x; 1.9367x over previous
import numpy as np
import jax
import jax.numpy as jnp
from jax.experimental import pallas as pl
from jax.experimental.pallas import tpu as pltpu


def _dct_mat(N):
    n = np.arange(N, dtype=np.float64)
    k = np.arange(N, dtype=np.float64)[:, None]
    M = np.cos(np.pi * (n + 0.5) * k / N)
    scale = np.where(k == 0, np.sqrt(1.0 / N), np.sqrt(2.0 / N))
    return M * scale


_A_NP = (_dct_mat(32).T @ _dct_mat(64)[:32, :]).astype(np.float32)


def _pool_body(a_ref, x_ref, o_ref):
    A = a_ref[...]
    xv = x_ref[0].reshape(64 * 64, 64)
    Y = jnp.concatenate(
        [jnp.dot(A, xv[d * 64:(d + 1) * 64], preferred_element_type=jnp.float32)
         for d in range(64)], axis=0)
    Y3 = Y[:, :32].reshape(64, 32, 32)
    o = jnp.einsum('kd,dlw->klw', A, Y3, preferred_element_type=jnp.float32)
    o_ref[0] = o


def kernel(x):
    B, C, D, H, W = x.shape
    BC = B * C
    xr = x.reshape(BC, D, H, W)
    A = jnp.asarray(_A_NP)

    out = pl.pallas_call(
        _pool_body,
        grid=(BC,),
        in_specs=[
            pl.BlockSpec((32, 64), lambda i: (0, 0)),
            pl.BlockSpec((1, 64, 64, 64), lambda i: (i, 0, 0, 0)),
        ],
        out_specs=pl.BlockSpec((1, 32, 32, 32), lambda i: (i, 0, 0, 0)),
        out_shape=jax.ShapeDtypeStruct((BC, 32, 32, 32), jnp.float32),
        compiler_params=pltpu.CompilerParams(
            dimension_semantics=("parallel",),
        ),
    )(A, xr)
    return out.reshape(B, C, 32, 32, 32)


# speedup vs baseline: 2.5167x; 1.2995x over previous
import numpy as np
import jax
import jax.numpy as jnp
from jax.experimental import pallas as pl
from jax.experimental.pallas import tpu as pltpu


def _dct_mat(N):
    n = np.arange(N, dtype=np.float64)
    k = np.arange(N, dtype=np.float64)[:, None]
    M = np.cos(np.pi * (n + 0.5) * k / N)
    scale = np.where(k == 0, np.sqrt(1.0 / N), np.sqrt(2.0 / N))
    return M * scale


_A_NP = (_dct_mat(32).T @ _dct_mat(64)[:32, :]).astype(np.float32)


_G = 2


def _pool_body(a_ref, x_ref, o_ref):
    A = a_ref[...]
    for g in range(_G):
        xv = x_ref[g].reshape(64 * 64, 64)
        Y = jnp.concatenate(
            [jnp.dot(A, xv[d * 64:(d + 1) * 64],
                     preferred_element_type=jnp.float32)
             for d in range(64)], axis=0)
        Y3 = Y[:, :32].reshape(64, 32, 32)
        o = jnp.einsum('kd,dlw->klw', A, Y3,
                       preferred_element_type=jnp.float32)
        o_ref[g] = o


def kernel(x):
    B, C, D, H, W = x.shape
    BC = B * C
    xr = x.reshape(BC, D, H, W)
    A = jnp.asarray(_A_NP)

    out = pl.pallas_call(
        _pool_body,
        grid=(BC // _G,),
        in_specs=[
            pl.BlockSpec((32, 64), lambda i: (0, 0)),
            pl.BlockSpec((_G, 64, 64, 64), lambda i: (i, 0, 0, 0)),
        ],
        out_specs=pl.BlockSpec((_G, 32, 32, 32), lambda i: (i, 0, 0, 0)),
        out_shape=jax.ShapeDtypeStruct((BC, 32, 32, 32), jnp.float32),
        compiler_params=pltpu.CompilerParams(
            dimension_semantics=("parallel",),
        ),
    )(A, xr)
    return out.reshape(B, C, 32, 32, 32)


# speedup vs baseline: 2.9748x; 1.1820x over previous
import numpy as np
import jax
import jax.numpy as jnp
from jax.experimental import pallas as pl
from jax.experimental.pallas import tpu as pltpu


def _dct_mat(N):
    n = np.arange(N, dtype=np.float64)
    k = np.arange(N, dtype=np.float64)[:, None]
    M = np.cos(np.pi * (n + 0.5) * k / N)
    scale = np.where(k == 0, np.sqrt(1.0 / N), np.sqrt(2.0 / N))
    return M * scale


_A_NP = (_dct_mat(32).T @ _dct_mat(64)[:32, :]).astype(np.float32)


_G = 4


def _pool_body(a_ref, x_ref, o_ref):
    A = a_ref[...]
    for g in range(_G):
        xv = x_ref[g].reshape(64 * 64, 64)
        Y = jnp.concatenate(
            [jnp.dot(A, xv[d * 64:(d + 1) * 64],
                     preferred_element_type=jnp.float32)
             for d in range(64)], axis=0)
        Y3 = Y[:, :32].reshape(64, 32, 32)
        o = jnp.einsum('kd,dlw->klw', A, Y3,
                       preferred_element_type=jnp.float32)
        o_ref[g] = o


def kernel(x):
    B, C, D, H, W = x.shape
    BC = B * C
    xr = x.reshape(BC, D, H, W)
    A = jnp.asarray(_A_NP)

    out = pl.pallas_call(
        _pool_body,
        grid=(BC // _G,),
        in_specs=[
            pl.BlockSpec((32, 64), lambda i: (0, 0)),
            pl.BlockSpec((_G, 64, 64, 64), lambda i: (i, 0, 0, 0)),
        ],
        out_specs=pl.BlockSpec((_G, 32, 32, 32), lambda i: (i, 0, 0, 0)),
        out_shape=jax.ShapeDtypeStruct((BC, 32, 32, 32), jnp.float32),
        compiler_params=pltpu.CompilerParams(
            dimension_semantics=("parallel",),
        ),
    )(A, xr)
    return out.reshape(B, C, 32, 32, 32)


# speedup vs baseline: 3.2402x; 1.0892x over previous
import numpy as np
import jax
import jax.numpy as jnp
from jax.experimental import pallas as pl
from jax.experimental.pallas import tpu as pltpu


def _dct_mat(N):
    n = np.arange(N, dtype=np.float64)
    k = np.arange(N, dtype=np.float64)[:, None]
    M = np.cos(np.pi * (n + 0.5) * k / N)
    scale = np.where(k == 0, np.sqrt(1.0 / N), np.sqrt(2.0 / N))
    return M * scale


_A_NP = (_dct_mat(32).T @ _dct_mat(64)[:32, :]).astype(np.float32)


_G = 8


def _pool_body(a_ref, x_ref, o_ref):
    A = a_ref[...]
    for g in range(_G):
        xv = x_ref[g].reshape(64 * 64, 64)[:, :32]
        Y = jnp.concatenate(
            [jnp.dot(A, xv[d * 64:(d + 1) * 64],
                     preferred_element_type=jnp.float32)
             for d in range(64)], axis=0)
        Y3 = Y.reshape(64, 32, 32)
        o = jnp.einsum('kd,dlw->klw', A, Y3,
                       preferred_element_type=jnp.float32)
        o_ref[g] = o


def kernel(x):
    B, C, D, H, W = x.shape
    BC = B * C
    xr = x.reshape(BC, D, H, W)
    A = jnp.asarray(_A_NP)

    out = pl.pallas_call(
        _pool_body,
        grid=(BC // _G,),
        in_specs=[
            pl.BlockSpec((32, 64), lambda i: (0, 0)),
            pl.BlockSpec((_G, 64, 64, 64), lambda i: (i, 0, 0, 0)),
        ],
        out_specs=pl.BlockSpec((_G, 32, 32, 32), lambda i: (i, 0, 0, 0)),
        out_shape=jax.ShapeDtypeStruct((BC, 32, 32, 32), jnp.float32),
        compiler_params=pltpu.CompilerParams(
            dimension_semantics=("parallel",),
            vmem_limit_bytes=56 * 1024 * 1024,
        ),
    )(A, xr)
    return out.reshape(B, C, 32, 32, 32)
